# baseline (speedup 1.0000x reference)
import jax
import jax.numpy as jnp
from jax import lax
from jax.experimental import pallas as pl
from jax.experimental.pallas import tpu as pltpu, tpu_sc as plsc

D = 64
DW = 128
NC, NS = 2, 16
NW = NC * NS
NX = 4

_MESH = plsc.VectorSubcoreMesh(
    core_axis_name="c", subcore_axis_name="s", num_cores=NC, num_subcores=NS
)


def _gather(batch: int, hist: int, vocab: int):
    xr = batch // NW
    nsuper = xr // NX
    hp = (hist + 7) // 8 * 8

    def body(x_hbm, table_hbm, out_hbm, idx_v, buf_a, buf_b, gsem, wsem):
        wid = lax.axis_index("c") * NS + lax.axis_index("s")
        base = wid * xr
        pltpu.sync_copy(x_hbm.at[pl.ds(base, xr)], idx_v)
        bufs = (buf_a, buf_b)

        def fire(g, buf):
            for k in range(NX):
                pltpu.async_copy(
                    table_hbm.at[idx_v.at[g * NX + k, pl.ds(0, hp)]],
                    buf.at[k],
                    gsem,
                )

        def drain(buf):
            for k in range(NX):
                pltpu.make_async_copy(
                    table_hbm.at[idx_v.at[k, pl.ds(0, hp)]],
                    buf.at[k],
                    gsem,
                ).wait()

        def write(g, buf):
            for k in range(NX):
                pltpu.async_copy(
                    buf.at[k],
                    out_hbm.at[base + g * NX + k],
                    wsem,
                )

        def wait_write(g, buf):
            for k in range(NX):
                pltpu.make_async_copy(
                    buf.at[k],
                    out_hbm.at[base + g * NX + k],
                    wsem,
                ).wait()

        fire(0, buf_a)

        def step(i, carry):
            for b in range(2):
                g = i * 2 + b
                nxt = bufs[1 - b]

                @pl.when(g >= 1)
                def _():
                    wait_write(g - 1, nxt)

                @pl.when(g + 1 < nsuper)
                def _():
                    fire(g + 1, nxt)

                drain(bufs[b])
                write(g, bufs[b])
            return carry

        lax.fori_loop(0, nsuper // 2, step, 0)
        wait_write(nsuper - 1, buf_b)

    return pl.kernel(
        body,
        out_type=jax.ShapeDtypeStruct((batch, hp, D), jnp.float32),
        mesh=_MESH,
        scratch_types=[
            pltpu.VMEM((xr, DW), jnp.int32),
            pltpu.VMEM((NX, hp, D), jnp.float32),
            pltpu.VMEM((NX, hp, D), jnp.float32),
            pltpu.SemaphoreType.DMA,
            pltpu.SemaphoreType.DMA,
        ],
        compiler_params=pltpu.CompilerParams(use_tc_tiling_on_sc=False),
    )


def kernel(x, table):
    b, h = x.shape
    v = table.shape[0]
    xp = jnp.pad(x, ((0, 0), (0, DW - h)))
    padded = _gather(b, h, v)(xp, table)
    return padded[:, :h, :]


# speedup vs baseline: 3.3374x; 3.3374x over previous
import jax
import jax.numpy as jnp
from jax import lax
from jax.experimental import pallas as pl
from jax.experimental.pallas import tpu as pltpu, tpu_sc as plsc

D = 64
DW = 128
NC, NS = 2, 16
NW = NC * NS
NX = 4

_MESH = plsc.VectorSubcoreMesh(
    core_axis_name="c", subcore_axis_name="s", num_cores=NC, num_subcores=NS
)
_PARAMS = pltpu.CompilerParams(use_tc_tiling_on_sc=True)


def _gather(batch: int, hist: int, vocab: int):
    xr = batch // NW
    nsuper = xr // NX
    hp = (hist + 7) // 8 * 8

    def body(x_hbm, wide_hbm, out_hbm, idx_v, buf_a, buf_b, gsem, wsem):
        wid = lax.axis_index("c") * NS + lax.axis_index("s")
        base = pl.multiple_of(wid * xr, 8)
        pltpu.sync_copy(x_hbm.at[pl.ds(base, xr)], idx_v)
        bufs = (buf_a, buf_b)

        def fire(g, buf):
            for k in range(NX):
                pltpu.async_copy(
                    wide_hbm.at[idx_v.at[g * NX + k]],
                    buf.at[k, pl.ds(0, hist)],
                    gsem,
                )

        def drain(buf):
            for k in range(NX):
                pltpu.make_async_copy(
                    wide_hbm.at[idx_v.at[k]],
                    buf.at[k, pl.ds(0, hist)],
                    gsem,
                ).wait()

        def write(g, buf):
            for k in range(NX):
                pltpu.async_copy(
                    buf.at[k], out_hbm.at[base + g * NX + k], wsem
                )

        def wait_write(g, buf):
            for k in range(NX):
                pltpu.make_async_copy(
                    buf.at[k], out_hbm.at[base + g * NX + k], wsem
                ).wait()

        fire(0, buf_a)

        def step(i, carry):
            for b in range(2):
                g = i * 2 + b
                nxt = bufs[1 - b]

                @pl.when(g >= 1)
                def _():
                    wait_write(g - 1, nxt)

                @pl.when(g + 1 < nsuper)
                def _():
                    fire(g + 1, nxt)

                drain(bufs[b])
                write(g, bufs[b])
            return carry

        lax.fori_loop(0, nsuper // 2, step, 0)
        wait_write(nsuper - 1, buf_b)

    return pl.kernel(
        body,
        out_type=jax.ShapeDtypeStruct((batch, hp, DW), jnp.float32),
        mesh=_MESH,
        scratch_types=[
            pltpu.VMEM((xr, hist), jnp.int32),
            pltpu.VMEM((NX, hp, DW), jnp.float32),
            pltpu.VMEM((NX, hp, DW), jnp.float32),
            pltpu.SemaphoreType.DMA,
            pltpu.SemaphoreType.DMA,
        ],
        compiler_params=_PARAMS,
    )


def kernel(x, table):
    b, h = x.shape
    v = table.shape[0]
    wide = jnp.pad(table, ((0, 0), (0, DW - D)))
    padded = _gather(b, h, v)(x, wide)
    return padded[:, :h, :D]
